# baseline (speedup 1.0000x reference)
import jax
import jax.numpy as jnp
from jax.experimental import pallas as pl

N_NODES = 10000
N_EDGES = 320000
N_GRAPHS = 16
D = 128


def _cnn_g(image_stack, Wconv, bconv, Wfc, bfc):
    xp = jnp.pad(image_stack, ((0, 0), (0, 0), (0, 2), (0, 2)))
    acc = jnp.zeros((image_stack.shape[0], 16, 224, 224), jnp.float32)
    for ky in range(3):
        for kx in range(3):
            acc = acc + jnp.einsum("oc,nchw->nohw", Wconv[:, :, ky, kx],
                                   xp[:, :, ky:ky + 224, kx:kx + 224])
    m = (jnp.arange(224) % 2 == 0).astype(jnp.float32)
    z = jax.nn.relu(acc + bconv[None, :, None, None])
    z = z * m[None, None, :, None] * m[None, None, None, :]
    y = jnp.sum(z, axis=(2, 3)) / (112.0 * 112.0)
    return y @ Wfc + bfc


def kernel(image_stack, node_categories, edge_categories, edge_connections,
           graph_idx_of_node, graph_idx_of_edge, node_table, edge_table,
           Wconv, bconv, Wfc, bfc,
           W_e0, b_e0, W_v0, b_v0, W_g0, b_g0,
           W_e1, b_e1, W_v1, b_v1, W_g1, b_g1,
           W_e2, b_e2, W_v2, b_v2, W_g2, b_g2):
    src, dst = edge_connections[0], edge_connections[1]
    gi_n = graph_idx_of_node
    gi_e = graph_idx_of_edge

    g = _cnn_g(image_stack, Wconv, bconv, Wfc, bfc)
    n = jnp.take(node_table, node_categories, axis=0)

    layers = [(W_e0, b_e0, W_v0, b_v0, W_g0, b_g0, 128),
              (W_e1, b_e1, W_v1, b_v1, W_g1, b_g1, 128),
              (W_e2, b_e2, W_v2, b_v2, W_g2, b_g2, 1)]

    t = None
    e_out = None
    for li, (We, be, Wv, bv, Wg, bg, eo) in enumerate(layers):
        ni = D
        We_e, We_s, We_d, We_g = We[:D], We[D:2 * D], We[2 * D:3 * D], We[3 * D:]
        Pns = n @ We_s
        Pnd = n @ We_d
        Pg = g @ We_g + be
        if li == 0:
            T0 = edge_table @ We_e
            t = jnp.take(T0, edge_categories, axis=0)
        z = t + Pns[src] + Pnd[dst] + Pg[gi_e]
        e_new = jax.nn.relu(z)
        agg_e_node = jax.ops.segment_sum(e_new, dst, num_segments=N_NODES)
        agg_e = jax.ops.segment_sum(e_new, gi_e, num_segments=N_GRAPHS)
        Wv_n, Wv_a, Wv_g = Wv[:ni], Wv[ni:ni + eo], Wv[ni + eo:]
        n_new = jax.nn.relu(n @ Wv_n + agg_e_node @ Wv_a + (g @ Wv_g)[gi_n] + bv)
        agg_n = jax.ops.segment_sum(n_new, gi_n, num_segments=N_GRAPHS)
        no = n_new.shape[1]
        Wg_g, Wg_n, Wg_e = Wg[:D], Wg[D:D + no], Wg[D + no:]
        g = jax.nn.relu(g @ Wg_g + agg_n @ Wg_n + agg_e @ Wg_e + bg)
        n = n_new
        e_out = e_new
        if li < 2:
            We_next = layers[li + 1][0]
            t = e_new @ We_next[:D]
    return (g, n, e_out)


# speedup vs baseline: 1.7131x; 1.7131x over previous
import functools

import jax
import jax.numpy as jnp
from jax import lax
from jax.experimental import pallas as pl
from jax.experimental.pallas import tpu as pltpu
from jax.experimental.pallas import tpu_sc as plsc

N_NODES = 10000
N_EDGES = 320000
NG = 16
D = 128

NC = 2
NS = 16
NW = NC * NS
EPT = N_EDGES // NW
CH = 80
NCH = EPT // CH
NPT = 624
NTAIL = N_NODES - NPT * NS

_mesh = plsc.VectorSubcoreMesh(core_axis_name="c", subcore_axis_name="s")


def _make_edge_pass(gather_t):
    def body(t_hbm, cati, srci, dsti, gii, pns, pnd, pg, zeros_hbm,
             e_out, parts_out,
             idx_t, idx_s, idx_d, idx_g, acc, buf_s, buf_d, buf_g, spm, sem):
        cid = lax.axis_index("c")
        sid = lax.axis_index("s")
        wid = sid * NC + cid
        pltpu.sync_copy(zeros_hbm, spm.at[pl.ds(sid * NPT, NPT)])

        @pl.when(sid == NS - 1)
        def _zero_tail():
            pltpu.sync_copy(zeros_hbm.at[pl.ds(0, NTAIL)],
                            spm.at[pl.ds(NPT * NS, NTAIL)])

        plsc.subcore_barrier()

        @pl.loop(0, NCH)
        def _chunk(c):
            base = wid * EPT + c * CH
            pltpu.sync_copy(srci.at[pl.ds(base, CH)], idx_s)
            pltpu.sync_copy(dsti.at[pl.ds(base, CH)], idx_d)
            pltpu.sync_copy(gii.at[pl.ds(base, CH)], idx_g)
            if gather_t:
                pltpu.sync_copy(cati.at[pl.ds(base, CH)], idx_t)
                pltpu.async_copy(t_hbm.at[idx_t], acc, sem).wait()
            else:
                pltpu.sync_copy(t_hbm.at[pl.ds(base, CH)], acc)
            pltpu.async_copy(pns.at[idx_s], buf_s, sem).wait()
            pltpu.async_copy(pnd.at[idx_d], buf_d, sem).wait()
            pltpu.async_copy(pg.at[idx_g], buf_g, sem).wait()

            @pl.loop(0, CH, unroll=2)
            def _row(r):
                for j in range(D // 16):
                    sl = pl.ds(j * 16, 16)
                    z = (acc[r, sl] + buf_s[r, sl] + buf_d[r, sl]
                         + buf_g[r, sl])
                    acc[r, sl] = jnp.maximum(z, 0.0)

            pltpu.sync_copy(acc, e_out.at[pl.ds(base, CH)])
            pltpu.sync_copy(acc, spm.at[idx_d], add=True)

        plsc.subcore_barrier()
        pltpu.sync_copy(spm.at[pl.ds(sid * NPT, NPT)],
                        parts_out.at[cid, pl.ds(sid * NPT, NPT)])

        @pl.when(sid == NS - 1)
        def _read_tail():
            pltpu.sync_copy(spm.at[pl.ds(NPT * NS, NTAIL)],
                            parts_out.at[cid, pl.ds(NPT * NS, NTAIL)])

    return pl.kernel(
        body,
        mesh=_mesh,
        out_type=(
            jax.ShapeDtypeStruct((N_EDGES, D), jnp.float32),
            jax.ShapeDtypeStruct((NC, N_NODES, D), jnp.float32),
        ),
        scratch_types=(
            [pltpu.VMEM((CH,), jnp.int32)] * 4
            + [pltpu.VMEM((CH, D), jnp.float32)] * 4
            + [pltpu.VMEM_SHARED((N_NODES, D), jnp.float32)]
            + [pltpu.SemaphoreType.DMA]
        ),
    )


_edge_pass_gather = _make_edge_pass(True)
_edge_pass_linear = _make_edge_pass(False)


def _l2_body(t2, srci, dsti, gii, pns, pnd, pg, zeros1d,
             e_out, nacc_out, gacc_out,
             tbuf, sbuf, dbuf, gbuf, pnsb, pndb, pgb, ebuf, naccb, gaccb):
    wid = lax.axis_index("s") * NC + lax.axis_index("c")
    base = wid * EPT
    pltpu.sync_copy(t2.at[pl.ds(base, EPT)], tbuf)
    pltpu.sync_copy(srci.at[pl.ds(base, EPT)], sbuf)
    pltpu.sync_copy(dsti.at[pl.ds(base, EPT)], dbuf)
    pltpu.sync_copy(gii.at[pl.ds(base, EPT)], gbuf)
    pltpu.sync_copy(pns, pnsb)
    pltpu.sync_copy(pnd, pndb)
    pltpu.sync_copy(pg, pgb)
    pltpu.sync_copy(zeros1d, naccb)
    gaccb[pl.ds(0, 16)] = jnp.zeros((16,), jnp.float32)

    @pl.loop(0, EPT // 16)
    def _vec(i):
        sl = pl.ds(i * 16, 16)
        s = sbuf[sl]
        d = dbuf[sl]
        gv = gbuf[sl]
        z = (tbuf[sl]
             + plsc.load_gather(pnsb, [s])
             + plsc.load_gather(pndb, [d])
             + plsc.load_gather(pgb, [gv]))
        en = jnp.maximum(z, 0.0)
        ebuf[sl] = en
        plsc.addupdate_scatter(naccb, [d], en)
        plsc.addupdate_scatter(gaccb, [gv], en)

    pltpu.sync_copy(ebuf, e_out.at[pl.ds(base, EPT)])
    pltpu.sync_copy(naccb, nacc_out.at[wid])
    pltpu.sync_copy(gaccb, gacc_out.at[wid])


_l2_edge = pl.kernel(
    _l2_body,
    mesh=_mesh,
    compiler_params=pltpu.CompilerParams(needs_layout_passes=False),
    out_type=(
        jax.ShapeDtypeStruct((N_EDGES,), jnp.float32),
        jax.ShapeDtypeStruct((NW, N_NODES), jnp.float32),
        jax.ShapeDtypeStruct((NW, 16), jnp.float32),
    ),
    scratch_types=(
        [pltpu.VMEM((EPT,), jnp.float32)]
        + [pltpu.VMEM((EPT,), jnp.int32)] * 3
        + [pltpu.VMEM((N_NODES,), jnp.float32)] * 2
        + [pltpu.VMEM((16,), jnp.float32)]
        + [pltpu.VMEM((EPT,), jnp.float32)]
        + [pltpu.VMEM((N_NODES,), jnp.float32)]
        + [pltpu.VMEM((16,), jnp.float32)]
    ),
)


def _cnn_g(image_stack, Wconv, bconv, Wfc, bfc):
    xp = jnp.pad(image_stack, ((0, 0), (0, 0), (0, 2), (0, 2)))
    acc = jnp.zeros((image_stack.shape[0], 16, 224, 224), jnp.float32)
    for ky in range(3):
        for kx in range(3):
            acc = acc + jnp.einsum("oc,nchw->nohw", Wconv[:, :, ky, kx],
                                   xp[:, :, ky:ky + 224, kx:kx + 224])
    m = (jnp.arange(224) % 2 == 0).astype(jnp.float32)
    z = jax.nn.relu(acc + bconv[None, :, None, None])
    z = z * m[None, None, :, None] * m[None, None, None, :]
    y = jnp.sum(z, axis=(2, 3)) / (112.0 * 112.0)
    return y @ Wfc + bfc


def kernel(image_stack, node_categories, edge_categories, edge_connections,
           graph_idx_of_node, graph_idx_of_edge, node_table, edge_table,
           Wconv, bconv, Wfc, bfc,
           W_e0, b_e0, W_v0, b_v0, W_g0, b_g0,
           W_e1, b_e1, W_v1, b_v1, W_g1, b_g1,
           W_e2, b_e2, W_v2, b_v2, W_g2, b_g2):
    src = edge_connections[0]
    dst = edge_connections[1]
    gi_n = graph_idx_of_node
    gi_e = graph_idx_of_edge
    zeros_np = jnp.zeros((NPT, D), jnp.float32)
    zeros_1d = jnp.zeros((N_NODES,), jnp.float32)

    g = _cnn_g(image_stack, Wconv, bconv, Wfc, bfc)
    n = jnp.take(node_table, node_categories, axis=0)

    layers = [(W_e0, b_e0, W_v0, b_v0, W_g0, b_g0),
              (W_e1, b_e1, W_v1, b_v1, W_g1, b_g1),
              (W_e2, b_e2, W_v2, b_v2, W_g2, b_g2)]

    t = None
    for li in (0, 1):
        We, be, Wv, bv, Wg, bg = layers[li]
        Pns = n @ We[D:2 * D]
        Pnd = n @ We[2 * D:3 * D]
        Pg = g @ We[3 * D:] + be
        if li == 0:
            T0 = edge_table @ We[:D]
            e_new, parts = _edge_pass_gather(T0, edge_categories, src, dst,
                                             gi_e, Pns, Pnd, Pg, zeros_np)
        else:
            e_new, parts = _edge_pass_linear(t, edge_categories, src, dst,
                                             gi_e, Pns, Pnd, Pg, zeros_np)
        agg_e_node = parts[0] + parts[1]
        agg_e = jax.ops.segment_sum(e_new, gi_e, num_segments=NG)
        n_new = jax.nn.relu(n @ Wv[:D] + agg_e_node @ Wv[D:D + D]
                            + (g @ Wv[2 * D:])[gi_n] + bv)
        agg_n = jax.ops.segment_sum(n_new, gi_n, num_segments=NG)
        g = jax.nn.relu(g @ Wg[:D] + agg_n @ Wg[D:2 * D]
                        + agg_e @ Wg[2 * D:] + bg)
        n = n_new
        t = e_new @ layers[li + 1][0][:D]

    We, be, Wv, bv, Wg, bg = layers[2]
    Pns2 = (n @ We[D:2 * D]).reshape(-1)
    Pnd2 = (n @ We[2 * D:3 * D]).reshape(-1)
    Pg2 = (g @ We[3 * D:] + be).reshape(-1)
    t2 = t.reshape(-1)
    e2, nacc, gacc = _l2_edge(t2, src, dst, gi_e, Pns2, Pnd2, Pg2, zeros_1d)
    agg_e_node2 = jnp.sum(nacc, axis=0).reshape(N_NODES, 1)
    agg_e2 = jnp.sum(gacc, axis=0).reshape(NG, 1)
    n_final = jax.nn.relu(n @ Wv[:D] + agg_e_node2 * Wv[D, 0]
                          + (g @ Wv[D + 1:])[gi_n] + bv)
    agg_n2 = jax.ops.segment_sum(n_final, gi_n, num_segments=NG)
    g_final = jax.nn.relu(g @ Wg[:D] + agg_n2 @ Wg[D:D + 1]
                          + agg_e2 @ Wg[D + 1:] + bg)
    return (g_final, n_final, e2.reshape(N_EDGES, 1))


# speedup vs baseline: 3.1503x; 1.8390x over previous
import functools

import jax
import jax.numpy as jnp
from jax import lax
from jax.experimental import pallas as pl
from jax.experimental.pallas import tpu as pltpu
from jax.experimental.pallas import tpu_sc as plsc

N_NODES = 10000
N_EDGES = 320000
NG = 16
D = 128

NC = 2
NS = 16
NW = NC * NS
EPT = N_EDGES // NW
CH = 40
NCH = EPT // CH
NPT = 624
NTAIL = N_NODES - NPT * NS

_mesh = plsc.VectorSubcoreMesh(core_axis_name="c", subcore_axis_name="s")


def _make_edge_pass(gather_t):
    def body(t_hbm, cati, srci, dsti, gii, pns, pnd, pg, zeros_hbm,
             e_out, parts_out,
             it0, it1, is0, is1, id0, id1, ig0, ig1, io0, io1,
             acc0, acc1, bs0, bs1, bd0, bd1, bg0, bg1,
             spm,
             si0, si1, sio0, sio1, st0, st1, sg0, sg1, so0, so1,
             sp0, sp1):
        IT = (it0, it1)
        IS = (is0, is1)
        ID = (id0, id1)
        IG = (ig0, ig1)
        IO = (io0, io1)
        ACC = (acc0, acc1)
        BS = (bs0, bs1)
        BD = (bd0, bd1)
        BG = (bg0, bg1)
        SI = (si0, si1)
        SIO = (sio0, sio1)
        ST = (st0, st1)
        SG = (sg0, sg1)
        SO = (so0, so1)
        SP = (sp0, sp1)
        cid = lax.axis_index("c")
        sid = lax.axis_index("s")
        wid = sid * NC + cid
        tile_base = wid * EPT

        def fire_idx(c, b):
            sl = pl.ds(tile_base + c * CH, CH)
            pltpu.async_copy(srci.at[sl], IS[b], SI[b])
            pltpu.async_copy(dsti.at[sl], ID[b], SI[b])
            pltpu.async_copy(gii.at[sl], IG[b], SI[b])
            if gather_t:
                pltpu.async_copy(cati.at[sl], IT[b], SI[b])

        def wait_idx(b):
            n = 4 if gather_t else 3
            for _ in range(n):
                pltpu.make_async_copy(
                    srci.at[pl.ds(0, CH)], IS[b], SI[b]).wait()

        def fire_ido(c, b):
            sl = pl.ds(tile_base + c * CH, CH)
            pltpu.async_copy(dsti.at[sl], IO[b], SIO[b])

        def wait_ido(b):
            pltpu.make_async_copy(
                dsti.at[pl.ds(0, CH)], IO[b], SIO[b]).wait()

        def fire_gathers(c, b):
            if gather_t:
                pltpu.async_copy(t_hbm.at[IT[b]], ACC[b], ST[b])
            else:
                pltpu.async_copy(
                    t_hbm.at[pl.ds(tile_base + c * CH, CH)], ACC[b], ST[b])
            pltpu.async_copy(pns.at[IS[b]], BS[b], SG[b])
            pltpu.async_copy(pnd.at[ID[b]], BD[b], SG[b])
            pltpu.async_copy(pg.at[IG[b]], BG[b], SG[b])

        def wait_gathers(b):
            if gather_t:
                pltpu.make_async_copy(t_hbm.at[IT[b]], ACC[b], ST[b]).wait()
            else:
                pltpu.make_async_copy(
                    t_hbm.at[pl.ds(tile_base, CH)], ACC[b], ST[b]).wait()
            for dst in (BS[b], BD[b], BG[b]):
                pltpu.make_async_copy(pns.at[IS[b]], dst, SG[b]).wait()

        def compute(b):
            accb, bsb, bdb, bgb = ACC[b], BS[b], BD[b], BG[b]

            @pl.loop(0, CH, unroll=4)
            def _row(r):
                for j in range(D // 16):
                    sl = pl.ds(j * 16, 16)
                    z = (accb[r, sl] + bsb[r, sl] + bdb[r, sl]
                         + bgb[r, sl])
                    accb[r, sl] = jnp.maximum(z, 0.0)

        def fire_out(c, b):
            sl = pl.ds(tile_base + c * CH, CH)
            pltpu.async_copy(ACC[b], e_out.at[sl], SO[b])
            pltpu.async_copy(ACC[b], spm.at[IO[b]], SP[b], add=True)

        def wait_out(b):
            pltpu.make_async_copy(
                ACC[b], e_out.at[pl.ds(tile_base, CH)], SO[b]).wait()
            pltpu.make_async_copy(ACC[b], spm.at[IO[b]], SP[b]).wait()

        def chunk_body(c, b):
            nb = 1 - b
            wait_gathers(b)

            @pl.when(c + 2 < NCH)
            def _():
                fire_idx(c + 2, b)

            fire_ido(c, b)

            @pl.when(c >= 1)
            def _():
                wait_out(nb)

            @pl.when(c + 1 < NCH)
            def _():
                wait_idx(nb)
                fire_gathers(c + 1, nb)

            compute(b)
            wait_ido(b)
            fire_out(c, b)

        pltpu.sync_copy(zeros_hbm, spm.at[pl.ds(sid * NPT, NPT)])

        @pl.when(sid == NS - 1)
        def _zero_tail():
            pltpu.sync_copy(zeros_hbm.at[pl.ds(0, NTAIL)],
                            spm.at[pl.ds(NPT * NS, NTAIL)])

        plsc.subcore_barrier()

        fire_idx(0, 0)
        fire_idx(1, 1)
        wait_idx(0)
        fire_gathers(0, 0)

        @pl.loop(0, NCH // 2)
        def _pair(i):
            chunk_body(2 * i, 0)
            chunk_body(2 * i + 1, 1)

        wait_out(1)

        plsc.subcore_barrier()
        pltpu.sync_copy(spm.at[pl.ds(sid * NPT, NPT)],
                        parts_out.at[cid, pl.ds(sid * NPT, NPT)])

        @pl.when(sid == NS - 1)
        def _read_tail():
            pltpu.sync_copy(spm.at[pl.ds(NPT * NS, NTAIL)],
                            parts_out.at[cid, pl.ds(NPT * NS, NTAIL)])

    return pl.kernel(
        body,
        mesh=_mesh,
        out_type=(
            jax.ShapeDtypeStruct((N_EDGES, D), jnp.float32),
            jax.ShapeDtypeStruct((NC, N_NODES, D), jnp.float32),
        ),
        scratch_types=(
            [pltpu.VMEM((CH,), jnp.int32)] * 10
            + [pltpu.VMEM((CH, D), jnp.float32)] * 8
            + [pltpu.VMEM_SHARED((N_NODES, D), jnp.float32)]
            + [pltpu.SemaphoreType.DMA] * 12
        ),
    )


_edge_pass_gather = _make_edge_pass(True)
_edge_pass_linear = _make_edge_pass(False)


EB = 3200
NEB = N_EDGES // EB


def _make_edge_matmul(wout):
    def body(e_ref, gi_ref, w_ref, t_ref, agg_ref, acc_ref):
        i = pl.program_id(0)

        @pl.when(i == 0)
        def _():
            acc_ref[...] = jnp.zeros((NG, D), jnp.float32)

        e = e_ref[...]
        t_ref[...] = lax.dot_general(
            e, w_ref[...], (((1,), (0,)), ((), ())),
            precision=lax.Precision.HIGHEST,
            preferred_element_type=jnp.float32)
        ids = gi_ref[0]
        ohT = (lax.broadcasted_iota(jnp.int32, (NG, EB), 0)
               == ids).astype(jnp.float32)
        acc_ref[...] += lax.dot_general(
            ohT, e, (((1,), (0,)), ((), ())),
            precision=lax.Precision.HIGHEST,
            preferred_element_type=jnp.float32)
        agg_ref[...] = acc_ref[...]

    return pl.pallas_call(
        body,
        grid=(NEB,),
        in_specs=[
            pl.BlockSpec((EB, D), lambda i: (i, 0)),
            pl.BlockSpec((1, 1, EB), lambda i: (i, 0, 0)),
            pl.BlockSpec((D, wout), lambda i: (0, 0)),
        ],
        out_specs=[
            pl.BlockSpec((EB, wout), lambda i: (i, 0)),
            pl.BlockSpec((NG, D), lambda i: (0, 0)),
        ],
        out_shape=[
            jax.ShapeDtypeStruct((N_EDGES, wout), jnp.float32),
            jax.ShapeDtypeStruct((NG, D), jnp.float32),
        ],
        scratch_shapes=[pltpu.VMEM((NG, D), jnp.float32)],
    )


_edge_matmul_full = _make_edge_matmul(D)
_edge_matmul_pad8 = _make_edge_matmul(8)


def _l2_body(t2, srci, dsti, gii, pns, pnd, pg, zeros1d,
             e_out, nacc_out, gacc_out,
             tbuf, sbuf, dbuf, gbuf, pnsb, pndb, pgb, ebuf, naccb, gaccb):
    wid = lax.axis_index("s") * NC + lax.axis_index("c")
    base = wid * EPT
    pltpu.sync_copy(t2.at[pl.ds(base, EPT)], tbuf)
    pltpu.sync_copy(srci.at[pl.ds(base, EPT)], sbuf)
    pltpu.sync_copy(dsti.at[pl.ds(base, EPT)], dbuf)
    pltpu.sync_copy(gii.at[pl.ds(base, EPT)], gbuf)
    pltpu.sync_copy(pns, pnsb)
    pltpu.sync_copy(pnd, pndb)
    pltpu.sync_copy(pg, pgb)
    pltpu.sync_copy(zeros1d, naccb)
    gaccb[pl.ds(0, 16)] = jnp.zeros((16,), jnp.float32)

    @pl.loop(0, EPT // 16)
    def _vec(i):
        sl = pl.ds(i * 16, 16)
        s = sbuf[sl]
        d = dbuf[sl]
        gv = gbuf[sl]
        z = (tbuf[sl]
             + plsc.load_gather(pnsb, [s])
             + plsc.load_gather(pndb, [d])
             + plsc.load_gather(pgb, [gv]))
        en = jnp.maximum(z, 0.0)
        ebuf[sl] = en
        plsc.addupdate_scatter(naccb, [d], en)
        plsc.addupdate_scatter(gaccb, [gv], en)

    pltpu.sync_copy(ebuf, e_out.at[pl.ds(base, EPT)])
    pltpu.sync_copy(naccb, nacc_out.at[wid])
    pltpu.sync_copy(gaccb, gacc_out.at[wid])


_l2_edge = pl.kernel(
    _l2_body,
    mesh=_mesh,
    compiler_params=pltpu.CompilerParams(needs_layout_passes=False),
    out_type=(
        jax.ShapeDtypeStruct((N_EDGES,), jnp.float32),
        jax.ShapeDtypeStruct((NW, N_NODES), jnp.float32),
        jax.ShapeDtypeStruct((NW, 16), jnp.float32),
    ),
    scratch_types=(
        [pltpu.VMEM((EPT,), jnp.float32)]
        + [pltpu.VMEM((EPT,), jnp.int32)] * 3
        + [pltpu.VMEM((N_NODES,), jnp.float32)] * 2
        + [pltpu.VMEM((16,), jnp.float32)]
        + [pltpu.VMEM((EPT,), jnp.float32)]
        + [pltpu.VMEM((N_NODES,), jnp.float32)]
        + [pltpu.VMEM((16,), jnp.float32)]
    ),
)


def _cnn_g(image_stack, Wconv, bconv, Wfc, bfc):
    xp = jnp.pad(image_stack, ((0, 0), (0, 0), (0, 2), (0, 2)))
    acc = jnp.zeros((image_stack.shape[0], 16, 224, 224), jnp.float32)
    for ky in range(3):
        for kx in range(3):
            acc = acc + jnp.einsum("oc,nchw->nohw", Wconv[:, :, ky, kx],
                                   xp[:, :, ky:ky + 224, kx:kx + 224])
    m = (jnp.arange(224) % 2 == 0).astype(jnp.float32)
    z = jax.nn.relu(acc + bconv[None, :, None, None])
    z = z * m[None, None, :, None] * m[None, None, None, :]
    y = jnp.sum(z, axis=(2, 3)) / (112.0 * 112.0)
    return y @ Wfc + bfc


def kernel(image_stack, node_categories, edge_categories, edge_connections,
           graph_idx_of_node, graph_idx_of_edge, node_table, edge_table,
           Wconv, bconv, Wfc, bfc,
           W_e0, b_e0, W_v0, b_v0, W_g0, b_g0,
           W_e1, b_e1, W_v1, b_v1, W_g1, b_g1,
           W_e2, b_e2, W_v2, b_v2, W_g2, b_g2):
    src = edge_connections[0]
    dst = edge_connections[1]
    gi_n = graph_idx_of_node
    gi_e = graph_idx_of_edge
    zeros_np = jnp.zeros((NPT, D), jnp.float32)
    zeros_1d = jnp.zeros((N_NODES,), jnp.float32)
    gi_e3 = gi_e.reshape(NEB, 1, EB)

    g = _cnn_g(image_stack, Wconv, bconv, Wfc, bfc)
    n = jnp.take(node_table, node_categories, axis=0)

    layers = [(W_e0, b_e0, W_v0, b_v0, W_g0, b_g0),
              (W_e1, b_e1, W_v1, b_v1, W_g1, b_g1),
              (W_e2, b_e2, W_v2, b_v2, W_g2, b_g2)]

    t = None
    for li in (0, 1):
        We, be, Wv, bv, Wg, bg = layers[li]
        Pns = n @ We[D:2 * D]
        Pnd = n @ We[2 * D:3 * D]
        Pg = g @ We[3 * D:] + be
        if li == 0:
            T0 = edge_table @ We[:D]
            e_new, parts = _edge_pass_gather(T0, edge_categories, src, dst,
                                             gi_e, Pns, Pnd, Pg, zeros_np)
        else:
            e_new, parts = _edge_pass_linear(t, edge_categories, src, dst,
                                             gi_e, Pns, Pnd, Pg, zeros_np)
        agg_e_node = parts[0] + parts[1]
        We_next = layers[li + 1][0][:D]
        if li == 0:
            t, agg_e = _edge_matmul_full(e_new, gi_e3, We_next)
        else:
            t, agg_e = _edge_matmul_pad8(
                e_new, gi_e3, jnp.pad(We_next, ((0, 0), (0, 7))))
        n_new = jax.nn.relu(n @ Wv[:D] + agg_e_node @ Wv[D:D + D]
                            + (g @ Wv[2 * D:])[gi_n] + bv)
        agg_n = jax.ops.segment_sum(n_new, gi_n, num_segments=NG)
        g = jax.nn.relu(g @ Wg[:D] + agg_n @ Wg[D:2 * D]
                        + agg_e @ Wg[2 * D:] + bg)
        n = n_new

    We, be, Wv, bv, Wg, bg = layers[2]
    Pns2 = (n @ We[D:2 * D]).reshape(-1)
    Pnd2 = (n @ We[2 * D:3 * D]).reshape(-1)
    Pg2 = (g @ We[3 * D:] + be).reshape(-1)
    t2 = t[:, 0]
    e2, nacc, gacc = _l2_edge(t2, src, dst, gi_e, Pns2, Pnd2, Pg2, zeros_1d)
    agg_e_node2 = jnp.sum(nacc, axis=0).reshape(N_NODES, 1)
    agg_e2 = jnp.sum(gacc, axis=0).reshape(NG, 1)
    n_final = jax.nn.relu(n @ Wv[:D] + agg_e_node2 * Wv[D, 0]
                          + (g @ Wv[D + 1:])[gi_n] + bv)
    agg_n2 = jax.ops.segment_sum(n_final, gi_n, num_segments=NG)
    g_final = jax.nn.relu(g @ Wg[:D] + agg_n2 @ Wg[D:D + 1]
                          + agg_e2 @ Wg[D + 1:] + bg)
    return (g_final, n_final, e2.reshape(N_EDGES, 1))


# speedup vs baseline: 3.5491x; 1.1266x over previous
import functools

import jax
import jax.numpy as jnp
from jax import lax
from jax.experimental import pallas as pl
from jax.experimental.pallas import tpu as pltpu
from jax.experimental.pallas import tpu_sc as plsc

N_NODES = 10000
N_EDGES = 320000
NG = 16
D = 128

NC = 2
NS = 16
NW = NC * NS
EPT = N_EDGES // NW
CH = 40
NCH = EPT // CH
NPT = 624
NTAIL = N_NODES - NPT * NS

_mesh = plsc.VectorSubcoreMesh(core_axis_name="c", subcore_axis_name="s")


def _make_edge_pass(gather_t):
    def body(t_hbm, cati, srci, dsti, gii, pns, pnd, pg, zeros_hbm,
             e_out, parts_out,
             it0, it1, is0, is1, id0, id1, ig0, ig1, io0, io1,
             acc0, acc1, bs0, bs1, bd0, bd1, bg0, bg1,
             spm,
             si0, si1, sio0, sio1, st0, st1, sg0, sg1, so0, so1,
             sp0, sp1):
        IT = (it0, it1)
        IS = (is0, is1)
        ID = (id0, id1)
        IG = (ig0, ig1)
        IO = (io0, io1)
        ACC = (acc0, acc1)
        BS = (bs0, bs1)
        BD = (bd0, bd1)
        BG = (bg0, bg1)
        SI = (si0, si1)
        SIO = (sio0, sio1)
        ST = (st0, st1)
        SG = (sg0, sg1)
        SO = (so0, so1)
        SP = (sp0, sp1)
        cid = lax.axis_index("c")
        sid = lax.axis_index("s")
        wid = sid * NC + cid
        tile_base = wid * EPT

        def fire_idx(c, b):
            sl = pl.ds(tile_base + c * CH, CH)
            pltpu.async_copy(srci.at[sl], IS[b], SI[b])
            pltpu.async_copy(dsti.at[sl], ID[b], SI[b])
            pltpu.async_copy(gii.at[sl], IG[b], SI[b])
            if gather_t:
                pltpu.async_copy(cati.at[sl], IT[b], SI[b])

        def wait_idx(b):
            n = 4 if gather_t else 3
            for _ in range(n):
                pltpu.make_async_copy(
                    srci.at[pl.ds(0, CH)], IS[b], SI[b]).wait()

        def fire_ido(c, b):
            sl = pl.ds(tile_base + c * CH, CH)
            pltpu.async_copy(dsti.at[sl], IO[b], SIO[b])

        def wait_ido(b):
            pltpu.make_async_copy(
                dsti.at[pl.ds(0, CH)], IO[b], SIO[b]).wait()

        def fire_gathers(c, b):
            if gather_t:
                pltpu.async_copy(t_hbm.at[IT[b]], ACC[b], ST[b])
            else:
                pltpu.async_copy(
                    t_hbm.at[pl.ds(tile_base + c * CH, CH)], ACC[b], ST[b])
            pltpu.async_copy(pns.at[IS[b]], BS[b], SG[b])
            pltpu.async_copy(pnd.at[ID[b]], BD[b], SG[b])
            pltpu.async_copy(pg.at[IG[b]], BG[b], SG[b])

        def wait_gathers(b):
            if gather_t:
                pltpu.make_async_copy(t_hbm.at[IT[b]], ACC[b], ST[b]).wait()
            else:
                pltpu.make_async_copy(
                    t_hbm.at[pl.ds(tile_base, CH)], ACC[b], ST[b]).wait()
            for dst in (BS[b], BD[b], BG[b]):
                pltpu.make_async_copy(pns.at[IS[b]], dst, SG[b]).wait()

        def compute(b):
            accb, bsb, bdb, bgb = ACC[b], BS[b], BD[b], BG[b]

            @pl.loop(0, CH, unroll=4)
            def _row(r):
                for j in range(D // 16):
                    sl = pl.ds(j * 16, 16)
                    z = (accb[r, sl] + bsb[r, sl] + bdb[r, sl]
                         + bgb[r, sl])
                    accb[r, sl] = jnp.maximum(z, 0.0)

        def fire_out(c, b):
            sl = pl.ds(tile_base + c * CH, CH)
            pltpu.async_copy(ACC[b], e_out.at[sl], SO[b])
            pltpu.async_copy(ACC[b], spm.at[IO[b]], SP[b], add=True)

        def wait_out(b):
            pltpu.make_async_copy(
                ACC[b], e_out.at[pl.ds(tile_base, CH)], SO[b]).wait()
            pltpu.make_async_copy(ACC[b], spm.at[IO[b]], SP[b]).wait()

        def chunk_body(c, b):
            nb = 1 - b
            wait_gathers(b)

            @pl.when(c + 2 < NCH)
            def _():
                fire_idx(c + 2, b)

            fire_ido(c, b)

            @pl.when(c >= 1)
            def _():
                wait_out(nb)

            @pl.when(c + 1 < NCH)
            def _():
                wait_idx(nb)
                fire_gathers(c + 1, nb)

            compute(b)
            wait_ido(b)
            fire_out(c, b)

        pltpu.sync_copy(zeros_hbm, spm.at[pl.ds(sid * NPT, NPT)])

        @pl.when(sid == NS - 1)
        def _zero_tail():
            pltpu.sync_copy(zeros_hbm.at[pl.ds(0, NTAIL)],
                            spm.at[pl.ds(NPT * NS, NTAIL)])

        plsc.subcore_barrier()

        fire_idx(0, 0)
        fire_idx(1, 1)
        wait_idx(0)
        fire_gathers(0, 0)

        @pl.loop(0, NCH // 2)
        def _pair(i):
            chunk_body(2 * i, 0)
            chunk_body(2 * i + 1, 1)

        wait_out(1)

        plsc.subcore_barrier()
        pltpu.sync_copy(spm.at[pl.ds(sid * NPT, NPT)],
                        parts_out.at[cid, pl.ds(sid * NPT, NPT)])

        @pl.when(sid == NS - 1)
        def _read_tail():
            pltpu.sync_copy(spm.at[pl.ds(NPT * NS, NTAIL)],
                            parts_out.at[cid, pl.ds(NPT * NS, NTAIL)])

    return pl.kernel(
        body,
        mesh=_mesh,
        out_type=(
            jax.ShapeDtypeStruct((N_EDGES, D), jnp.float32),
            jax.ShapeDtypeStruct((NC, N_NODES, D), jnp.float32),
        ),
        scratch_types=(
            [pltpu.VMEM((CH,), jnp.int32)] * 10
            + [pltpu.VMEM((CH, D), jnp.float32)] * 8
            + [pltpu.VMEM_SHARED((N_NODES, D), jnp.float32)]
            + [pltpu.SemaphoreType.DMA] * 12
        ),
    )


_edge_pass_gather = _make_edge_pass(True)
_edge_pass_linear = _make_edge_pass(False)


EB = 3200
NEB = N_EDGES // EB


def _make_edge_matmul(wout):
    def body(e_ref, gi_ref, w_ref, t_ref, agg_ref, acc_ref):
        i = pl.program_id(0)

        @pl.when(i == 0)
        def _():
            acc_ref[...] = jnp.zeros((NG, D), jnp.float32)

        e = e_ref[...]
        t_ref[...] = lax.dot_general(
            e, w_ref[...], (((1,), (0,)), ((), ())),
            precision=lax.Precision.HIGHEST,
            preferred_element_type=jnp.float32)
        ids = gi_ref[0]
        ohT = (lax.broadcasted_iota(jnp.int32, (NG, EB), 0)
               == ids).astype(jnp.float32)
        acc_ref[...] += lax.dot_general(
            ohT, e, (((1,), (0,)), ((), ())),
            precision=lax.Precision.HIGHEST,
            preferred_element_type=jnp.float32)
        agg_ref[...] = acc_ref[...]

    return pl.pallas_call(
        body,
        grid=(NEB,),
        in_specs=[
            pl.BlockSpec((EB, D), lambda i: (i, 0)),
            pl.BlockSpec((1, 1, EB), lambda i: (i, 0, 0)),
            pl.BlockSpec((D, wout), lambda i: (0, 0)),
        ],
        out_specs=[
            pl.BlockSpec((EB, wout), lambda i: (i, 0)),
            pl.BlockSpec((NG, D), lambda i: (0, 0)),
        ],
        out_shape=[
            jax.ShapeDtypeStruct((N_EDGES, wout), jnp.float32),
            jax.ShapeDtypeStruct((NG, D), jnp.float32),
        ],
        scratch_shapes=[pltpu.VMEM((NG, D), jnp.float32)],
    )


_edge_matmul_full = _make_edge_matmul(D)
_edge_matmul_pad8 = _make_edge_matmul(8)


def _l2_body(t2, srci, dsti, gii, pns, pnd, pg, zeros1d,
             e_out, nacc_out, gacc_out,
             tbuf, sbuf, dbuf, gbuf, pnsb, pndb, pgb, ebuf, naccb, gaccb):
    wid = lax.axis_index("s") * NC + lax.axis_index("c")
    base = wid * EPT
    pltpu.sync_copy(t2.at[pl.ds(base, EPT)], tbuf)
    pltpu.sync_copy(srci.at[pl.ds(base, EPT)], sbuf)
    pltpu.sync_copy(dsti.at[pl.ds(base, EPT)], dbuf)
    pltpu.sync_copy(gii.at[pl.ds(base, EPT)], gbuf)
    pltpu.sync_copy(pns, pnsb)
    pltpu.sync_copy(pnd, pndb)
    pltpu.sync_copy(pg, pgb)
    pltpu.sync_copy(zeros1d, naccb)
    gaccb[pl.ds(0, 16)] = jnp.zeros((16,), jnp.float32)

    @pl.loop(0, EPT // 16)
    def _vec(i):
        sl = pl.ds(i * 16, 16)
        s = sbuf[sl]
        d = dbuf[sl]
        gv = gbuf[sl]
        z = (tbuf[sl]
             + plsc.load_gather(pnsb, [s])
             + plsc.load_gather(pndb, [d])
             + plsc.load_gather(pgb, [gv]))
        en = jnp.maximum(z, 0.0)
        ebuf[sl] = en
        plsc.addupdate_scatter(naccb, [d], en)
        plsc.addupdate_scatter(gaccb, [gv], en)

    pltpu.sync_copy(ebuf, e_out.at[pl.ds(base, EPT)])
    pltpu.sync_copy(naccb, nacc_out.at[wid])
    pltpu.sync_copy(gaccb, gacc_out.at[wid])


_l2_edge = pl.kernel(
    _l2_body,
    mesh=_mesh,
    compiler_params=pltpu.CompilerParams(needs_layout_passes=False),
    out_type=(
        jax.ShapeDtypeStruct((N_EDGES,), jnp.float32),
        jax.ShapeDtypeStruct((NW, N_NODES), jnp.float32),
        jax.ShapeDtypeStruct((NW, 16), jnp.float32),
    ),
    scratch_types=(
        [pltpu.VMEM((EPT,), jnp.float32)]
        + [pltpu.VMEM((EPT,), jnp.int32)] * 3
        + [pltpu.VMEM((N_NODES,), jnp.float32)] * 2
        + [pltpu.VMEM((16,), jnp.float32)]
        + [pltpu.VMEM((EPT,), jnp.float32)]
        + [pltpu.VMEM((N_NODES,), jnp.float32)]
        + [pltpu.VMEM((16,), jnp.float32)]
    ),
)


def _cnn_body(x_ref, w_ref, b_ref, s_ref):
    x = x_ref[0]
    w = w_ref[...]
    acc = jnp.zeros((16, 224, 224), jnp.float32)
    for ky in range(3):
        for kx in range(3):
            xs = x[:, ky:ky + 224, kx:kx + 224]
            for ci in range(3):
                acc = acc + w[:, ci, ky, kx][:, None, None] * xs[ci][None]
    z = jnp.maximum(acc + b_ref[...][:, None, None], 0.0)
    m2 = ((lax.broadcasted_iota(jnp.int32, (224, 224), 0) % 2 == 0)
          & (lax.broadcasted_iota(jnp.int32, (224, 224), 1) % 2 == 0))
    zm = jnp.where(m2[None], z, 0.0)
    s_ref[0, 0, :] = jnp.sum(zm, axis=(1, 2)) * (1.0 / (112.0 * 112.0))


_cnn_feats = pl.pallas_call(
    _cnn_body,
    grid=(NG,),
    in_specs=[
        pl.BlockSpec((1, 3, 226, 226), lambda i: (i, 0, 0, 0)),
        pl.BlockSpec((16, 3, 3, 3), lambda i: (0, 0, 0, 0)),
        pl.BlockSpec((16,), lambda i: (0,)),
    ],
    out_specs=pl.BlockSpec((1, 1, 16), lambda i: (i, 0, 0)),
    out_shape=jax.ShapeDtypeStruct((NG, 1, 16), jnp.float32),
)


NB = 5
NBS = N_NODES // NB


def _dotT(a, b):
    return lax.dot_general(a, b, (((0,), (0,)), ((), ())),
                           precision=lax.Precision.HIGHEST,
                           preferred_element_type=jnp.float32)


def _dot(a, b):
    return lax.dot_general(a, b, (((1,), (0,)), ((), ())),
                           precision=lax.Precision.HIGHEST,
                           preferred_element_type=jnp.float32)


def _pre_body(cats_ref, nt_ref, we_ref, s_ref, wfc_ref, bfc_ref, et_ref,
              be_ref, n0_ref, pns_ref, pnd_ref, g0_ref, t0_ref, pg_ref):
    cats = cats_ref[0]
    ohT = (lax.broadcasted_iota(jnp.int32, (32, NBS), 0)
           == cats).astype(jnp.float32)
    we = we_ref[...]
    n0 = _dotT(ohT, nt_ref[...])
    n0_ref[...] = n0
    pns_ref[...] = _dot(n0, we[D:2 * D])
    pnd_ref[...] = _dot(n0, we[2 * D:3 * D])
    g0 = _dot(s_ref[...], wfc_ref[...]) + bfc_ref[...][None]
    g0_ref[...] = g0
    t0_ref[...] = _dot(et_ref[...], we[:D])
    pg_ref[...] = _dot(g0, we[3 * D:]) + be_ref[...][None]


_precompute0 = pl.pallas_call(
    _pre_body,
    grid=(NB,),
    in_specs=[
        pl.BlockSpec((1, 1, NBS), lambda i: (i, 0, 0)),
        pl.BlockSpec((32, D), lambda i: (0, 0)),
        pl.BlockSpec((4 * D, D), lambda i: (0, 0)),
        pl.BlockSpec((NG, 16), lambda i: (0, 0)),
        pl.BlockSpec((16, D), lambda i: (0, 0)),
        pl.BlockSpec((D,), lambda i: (0,)),
        pl.BlockSpec((16, D), lambda i: (0, 0)),
        pl.BlockSpec((D,), lambda i: (0,)),
    ],
    out_specs=[
        pl.BlockSpec((NBS, D), lambda i: (i, 0)),
        pl.BlockSpec((NBS, D), lambda i: (i, 0)),
        pl.BlockSpec((NBS, D), lambda i: (i, 0)),
        pl.BlockSpec((NG, D), lambda i: (0, 0)),
        pl.BlockSpec((16, D), lambda i: (0, 0)),
        pl.BlockSpec((NG, D), lambda i: (0, 0)),
    ],
    out_shape=[
        jax.ShapeDtypeStruct((N_NODES, D), jnp.float32),
        jax.ShapeDtypeStruct((N_NODES, D), jnp.float32),
        jax.ShapeDtypeStruct((N_NODES, D), jnp.float32),
        jax.ShapeDtypeStruct((NG, D), jnp.float32),
        jax.ShapeDtypeStruct((16, D), jnp.float32),
        jax.ShapeDtypeStruct((NG, D), jnp.float32),
    ],
)


def _make_node_update(wout):
    def body(n_ref, parts_ref, gin_ref, wv_ref, bv_ref, g_ref, agge_ref,
             wg_ref, bg_ref, wen_ref, ben_ref,
             nn_ref, aggn_ref, gn_ref, pns_ref, pnd_ref, pgn_ref, accn_ref):
        i = pl.program_id(0)

        @pl.when(i == 0)
        def _():
            accn_ref[...] = jnp.zeros((NG, D), jnp.float32)

        agg = parts_ref[0] + parts_ref[1]
        gv = g_ref[...]
        wv = wv_ref[...]
        ids = gin_ref[0]
        ohT = (lax.broadcasted_iota(jnp.int32, (NG, NBS), 0)
               == ids).astype(jnp.float32)
        gterm = _dotT(ohT, _dot(gv, wv[2 * D:]))
        n_new = jnp.maximum(
            _dot(n_ref[...], wv[:D]) + _dot(agg, wv[D:2 * D]) + gterm
            + bv_ref[...][None], 0.0)
        nn_ref[...] = n_new
        accn_ref[...] += _dot(ohT, n_new)
        aggn_ref[...] = accn_ref[...]
        wg = wg_ref[...]
        g_new = jnp.maximum(
            _dot(gv, wg[:D]) + _dot(accn_ref[...], wg[D:2 * D])
            + _dot(agge_ref[...], wg[2 * D:]) + bg_ref[...][None], 0.0)
        gn_ref[...] = g_new
        wen = wen_ref[...]
        pns_ref[...] = _dot(n_new, wen[D:2 * D])
        pnd_ref[...] = _dot(n_new, wen[2 * D:3 * D])
        pgn_ref[...] = _dot(g_new, wen[3 * D:]) + ben_ref[...][None]

    return pl.pallas_call(
        body,
        grid=(NB,),
        in_specs=[
            pl.BlockSpec((NBS, D), lambda i: (i, 0)),
            pl.BlockSpec((NC, NBS, D), lambda i: (0, i, 0)),
            pl.BlockSpec((1, 1, NBS), lambda i: (i, 0, 0)),
            pl.BlockSpec((3 * D, D), lambda i: (0, 0)),
            pl.BlockSpec((D,), lambda i: (0,)),
            pl.BlockSpec((NG, D), lambda i: (0, 0)),
            pl.BlockSpec((NG, D), lambda i: (0, 0)),
            pl.BlockSpec((3 * D, D), lambda i: (0, 0)),
            pl.BlockSpec((D,), lambda i: (0,)),
            pl.BlockSpec((4 * D, wout), lambda i: (0, 0)),
            pl.BlockSpec((wout,), lambda i: (0,)),
        ],
        out_specs=[
            pl.BlockSpec((NBS, D), lambda i: (i, 0)),
            pl.BlockSpec((NG, D), lambda i: (0, 0)),
            pl.BlockSpec((NG, D), lambda i: (0, 0)),
            pl.BlockSpec((NBS, wout), lambda i: (i, 0)),
            pl.BlockSpec((NBS, wout), lambda i: (i, 0)),
            pl.BlockSpec((NG, wout), lambda i: (0, 0)),
        ],
        out_shape=[
            jax.ShapeDtypeStruct((N_NODES, D), jnp.float32),
            jax.ShapeDtypeStruct((NG, D), jnp.float32),
            jax.ShapeDtypeStruct((NG, D), jnp.float32),
            jax.ShapeDtypeStruct((N_NODES, wout), jnp.float32),
            jax.ShapeDtypeStruct((N_NODES, wout), jnp.float32),
            jax.ShapeDtypeStruct((NG, wout), jnp.float32),
        ],
        scratch_shapes=[pltpu.VMEM((NG, D), jnp.float32)],
    )


_node_update_full = _make_node_update(D)
_node_update_pad8 = _make_node_update(8)


def _final_body(n_ref, nacc_ref, gin_ref, wv_ref, bv_ref, g_ref, gacc_ref,
                wg_ref, bg_ref, nf_ref, gf_ref, accn_ref):
    i = pl.program_id(0)

    @pl.when(i == 0)
    def _():
        accn_ref[...] = jnp.zeros((NG, 8), jnp.float32)

    ones32 = jnp.ones((NW, 8), jnp.float32)
    agg2 = _dotT(nacc_ref[0], ones32)
    wv = wv_ref[...]
    gv = g_ref[...]
    ids = gin_ref[0]
    ohT = (lax.broadcasted_iota(jnp.int32, (NG, NBS), 0)
           == ids).astype(jnp.float32)
    gterm = _dotT(ohT, _dot(gv, wv[D + 1:2 * D + 1]))
    nf = jnp.maximum(
        _dot(n_ref[...], wv[:D]) + agg2 * wv[D:D + 1, :] + gterm
        + bv_ref[...][None], 0.0)
    nf_ref[...] = nf
    accn_ref[...] += lax.dot_general(
        ohT, nf, (((1,), (0,)), ((), ())),
        precision=lax.Precision.HIGHEST,
        preferred_element_type=jnp.float32)
    agge2 = _dotT(gacc_ref[...], jnp.ones((NW, 8), jnp.float32))
    wg = wg_ref[...]
    gf = jnp.maximum(
        _dot(gv, wg[:D]) + _dot(accn_ref[...][:, 0:1], wg[D:D + 1])
        + _dot(agge2[:, 0:1], wg[D + 1:D + 2]) + bg_ref[...][None], 0.0)
    gf_ref[...] = gf


_final_update = pl.pallas_call(
    _final_body,
    grid=(NB,),
    in_specs=[
        pl.BlockSpec((NBS, D), lambda i: (i, 0)),
        pl.BlockSpec((1, NW, NBS), lambda i: (i, 0, 0)),
        pl.BlockSpec((1, 1, NBS), lambda i: (i, 0, 0)),
        pl.BlockSpec((2 * D + 1, 8), lambda i: (0, 0)),
        pl.BlockSpec((8,), lambda i: (0,)),
        pl.BlockSpec((NG, D), lambda i: (0, 0)),
        pl.BlockSpec((NW, NG), lambda i: (0, 0)),
        pl.BlockSpec((D + 2, D), lambda i: (0, 0)),
        pl.BlockSpec((D,), lambda i: (0,)),
    ],
    out_specs=[
        pl.BlockSpec((NBS, 8), lambda i: (i, 0)),
        pl.BlockSpec((NG, D), lambda i: (0, 0)),
    ],
    out_shape=[
        jax.ShapeDtypeStruct((N_NODES, 8), jnp.float32),
        jax.ShapeDtypeStruct((NG, D), jnp.float32),
    ],
    scratch_shapes=[pltpu.VMEM((NG, 8), jnp.float32)],
)


def kernel(image_stack, node_categories, edge_categories, edge_connections,
           graph_idx_of_node, graph_idx_of_edge, node_table, edge_table,
           Wconv, bconv, Wfc, bfc,
           W_e0, b_e0, W_v0, b_v0, W_g0, b_g0,
           W_e1, b_e1, W_v1, b_v1, W_g1, b_g1,
           W_e2, b_e2, W_v2, b_v2, W_g2, b_g2):
    src = edge_connections[0]
    dst = edge_connections[1]
    gi_e = graph_idx_of_edge
    zeros_np = jnp.zeros((NPT, D), jnp.float32)
    zeros_1d = jnp.zeros((N_NODES,), jnp.float32)
    gi_e3 = gi_e.reshape(NEB, 1, EB)
    gi_n3 = graph_idx_of_node.reshape(NB, 1, NBS)
    cats3 = node_categories.reshape(NB, 1, NBS)
    xp = jnp.pad(image_stack, ((0, 0), (0, 0), (0, 2), (0, 2)))

    s_feats = _cnn_feats(xp, Wconv, bconv).reshape(NG, 16)
    n0, Pns0, Pnd0, g0, T0, Pg0 = _precompute0(
        cats3, node_table, W_e0, s_feats, Wfc, bfc, edge_table, b_e0)

    e0, parts0 = _edge_pass_gather(T0, edge_categories, src, dst, gi_e,
                                   Pns0, Pnd0, Pg0, zeros_np)
    t1, aggE0 = _edge_matmul_full(e0, gi_e3, W_e1[:D])
    n1, aggN0, g1, Pns1, Pnd1, Pg1 = _node_update_full(
        n0, parts0, gi_n3, W_v0, b_v0, g0, aggE0, W_g0, b_g0, W_e1, b_e1)

    e1, parts1 = _edge_pass_linear(t1, edge_categories, src, dst, gi_e,
                                   Pns1, Pnd1, Pg1, zeros_np)
    t2p, aggE1 = _edge_matmul_pad8(
        e1, gi_e3, jnp.pad(W_e2[:D], ((0, 0), (0, 7))))
    n2, aggN1, g2, Pns2p, Pnd2p, Pg2p = _node_update_pad8(
        n1, parts1, gi_n3, W_v1, b_v1, g1, aggE1, W_g1, b_g1,
        jnp.pad(W_e2, ((0, 0), (0, 7))), jnp.pad(b_e2, (0, 7)))

    e2, nacc, gacc = _l2_edge(t2p[:, 0], src, dst, gi_e,
                              Pns2p[:, 0], Pnd2p[:, 0], Pg2p[:, 0], zeros_1d)
    nacc3 = nacc.reshape(NW, NB, NBS).transpose(1, 0, 2)
    nf_pad, g_final = _final_update(
        n2, nacc3, gi_n3, jnp.pad(W_v2, ((0, 0), (0, 7))),
        jnp.pad(b_v2, (0, 7)), g2, gacc, W_g2, b_g2)
    return (g_final, nf_pad[:, 0:1], e2.reshape(N_EDGES, 1))
